# baseline (speedup 1.0000x reference)
import functools

import jax
import jax.numpy as jnp
from jax import lax
from jax.experimental import pallas as pl
from jax.experimental.pallas import tpu as pltpu
from jax.experimental.pallas import tpu_sc as plsc

EMBED = 64
_NC = 2
_NS = 16
_NW = _NC * _NS


@jax.jit
def _bpr_gather(user, item, user_table, item_table):
  B = user.shape[0]
  b_per_w = B // _NW

  @functools.partial(
      pl.kernel,
      mesh=plsc.VectorSubcoreMesh(core_axis_name="c", subcore_axis_name="s"),
      compiler_params=pltpu.CompilerParams(use_tc_tiling_on_sc=False),
      out_type=(
          jax.ShapeDtypeStruct((B, EMBED), jnp.float32),
          jax.ShapeDtypeStruct((B, EMBED), jnp.float32),
      ),
      scratch_types=[
          pltpu.VMEM((b_per_w,), jnp.int32),
          pltpu.VMEM((b_per_w,), jnp.int32),
          pltpu.VMEM((b_per_w, EMBED), jnp.float32),
          pltpu.VMEM((b_per_w, EMBED), jnp.float32),
          pltpu.SemaphoreType.DMA,
          pltpu.SemaphoreType.DMA,
      ],
  )
  def k(user_hbm, item_hbm, ut_hbm, it_hbm, uout_hbm, iout_hbm,
        uidx_v, iidx_v, urows_v, irows_v, usem, isem):
    wid = lax.axis_index("s") * _NC + lax.axis_index("c")
    base = wid * b_per_w
    pltpu.sync_copy(user_hbm.at[pl.ds(base, b_per_w)], uidx_v)
    pltpu.sync_copy(item_hbm.at[pl.ds(base, b_per_w)], iidx_v)
    ucopy = pltpu.async_copy(ut_hbm.at[uidx_v], urows_v, usem)
    icopy = pltpu.async_copy(it_hbm.at[iidx_v], irows_v, isem)
    ucopy.wait()
    pltpu.sync_copy(urows_v, uout_hbm.at[pl.ds(base, b_per_w)])
    icopy.wait()
    pltpu.sync_copy(irows_v, iout_hbm.at[pl.ds(base, b_per_w)])

  return k(user, item, user_table, item_table)


def kernel(user, item, user_table, item_table):
  return _bpr_gather(user.astype(jnp.int32), item.astype(jnp.int32),
                     user_table, item_table)


# speedup vs baseline: 2.2265x; 2.2265x over previous
import functools

import jax
import jax.numpy as jnp
from jax import lax
from jax.experimental import pallas as pl
from jax.experimental.pallas import tpu as pltpu
from jax.experimental.pallas import tpu_sc as plsc

EMBED = 64
_NC = 2
_NS = 16
_NW = _NC * _NS
_BANK = 4
_LANES = 128


@jax.jit
def _bpr_gather(user, item, ut_t, it_t):
  B = user.shape[0]
  bw = B // _NW
  groups = bw // 16

  @functools.partial(
      pl.kernel,
      mesh=plsc.VectorSubcoreMesh(core_axis_name="c", subcore_axis_name="s"),
      compiler_params=pltpu.CompilerParams(needs_layout_passes=False),
      out_type=(
          jax.ShapeDtypeStruct((B, EMBED), jnp.float32),
          jax.ShapeDtypeStruct((B, EMBED), jnp.float32),
      ),
      scratch_types=[
          pltpu.VMEM((bw,), jnp.int32),
          pltpu.VMEM((bw,), jnp.int32),
          pltpu.VMEM((2 * _BANK, EMBED, _LANES), jnp.float32),
          pltpu.VMEM((bw // 2, EMBED), jnp.float32),
      ] + [pltpu.SemaphoreType.DMA] * 2,
  )
  def k(uidx_hbm, iidx_hbm, ut_hbm, it_hbm, uout_hbm, iout_hbm,
        uidx_v, iidx_v, blk_v, outw_v, *sems):
    wid = lax.axis_index("s") * _NC + lax.axis_index("c")
    base = wid * bw
    pltpu.sync_copy(uidx_hbm.at[pl.ds(base, bw)], uidx_v)
    pltpu.sync_copy(iidx_hbm.at[pl.ds(base, bw)], iidx_v)

    def select(tab_blk, lane, row):
      lanes = jnp.zeros((16,), jnp.int32) + lane
      for g in range(EMBED // 16):
        rows = lax.iota(jnp.int32, 16) + g * 16
        v = plsc.load_gather(tab_blk, [rows, lanes])
        outw_v[row, pl.ds(g * 16, 16)] = v

    def do_table(tab_hbm, idx_v, out_hbm, h):
      def group(g16, _):
        ivec = idx_v[pl.ds((h * groups // 2 + g16) * 16, 16)]
        lanes_c = [None] * 16

        def fire(bank, c0):
          handles = []
          for j in range(_BANK):
            c = c0 + j
            i = ivec[c]
            start = pl.multiple_of((i // _LANES) * _LANES, _LANES)
            lanes_c[c] = i - start
            handles.append(pltpu.async_copy(
                tab_hbm.at[:, pl.ds(start, _LANES)],
                blk_v.at[bank * _BANK + j],
                sems[bank],
            ))
          return handles

        def drain_select(bank, c0, handles):
          for h in handles:
            h.wait()
          for j in range(_BANK):
            c = c0 + j
            select(blk_v.at[bank * _BANK + j], lanes_c[c], g16 * 16 + c)

        ha0 = fire(0, 0)
        hb0 = fire(1, 4)
        drain_select(0, 0, ha0)
        ha1 = fire(0, 8)
        drain_select(1, 4, hb0)
        hb1 = fire(1, 12)
        drain_select(0, 8, ha1)
        drain_select(1, 12, hb1)
        return 0

      lax.fori_loop(0, groups // 2, group, 0)
      pltpu.sync_copy(outw_v, out_hbm.at[pl.ds(base + h * (bw // 2), bw // 2)])

    do_table(ut_hbm, uidx_v, uout_hbm, 0)
    do_table(ut_hbm, uidx_v, uout_hbm, 1)
    do_table(it_hbm, iidx_v, iout_hbm, 0)
    do_table(it_hbm, iidx_v, iout_hbm, 1)

  return k(user, item, ut_t, it_t)


def kernel(user, item, user_table, item_table):
  user_e, item_e = _bpr_gather(
      user.astype(jnp.int32), item.astype(jnp.int32),
      user_table.T, item_table.T,
  )
  return (user_e, item_e)
